# baseline (speedup 1.0000x reference)
import jax
import jax.numpy as jnp
from jax import lax
from jax.experimental import pallas as pl
from jax.experimental.pallas import tpu as pltpu
from jax.experimental.pallas import tpu_sc as plsc

_NC = 2
_NS = 16
_NW = _NC * _NS
_CHUNK = 128
_NBUF = 2


def kernel(x, table):
    num_indices = x.shape[0] * x.shape[1]
    embed_dim = table.shape[1]
    n_per_w = num_indices // _NW
    n_chunks = n_per_w // _CHUNK
    indices = x.T.reshape(num_indices)

    mesh = plsc.VectorSubcoreMesh(core_axis_name="c", subcore_axis_name="s")

    @jax.jit
    @pl.kernel(
        out_type=jax.ShapeDtypeStruct((num_indices, embed_dim), jnp.float32),
        mesh=mesh,
        scratch_types=(
            [pltpu.VMEM((n_per_w,), jnp.int32)]
            + [pltpu.VMEM((_CHUNK, embed_dim), jnp.float32) for _ in range(_NBUF)]
            + [pltpu.SemaphoreType.DMA for _ in range(2 * _NBUF + 1)]
        ),
    )
    def gather_kernel(table_hbm, idx_hbm, out_hbm, idx_v, *bufs_and_sems):
        bufs = bufs_and_sems[:_NBUF]
        gsems = bufs_and_sems[_NBUF:2 * _NBUF]
        wsems = bufs_and_sems[2 * _NBUF:3 * _NBUF]
        isem = bufs_and_sems[3 * _NBUF]

        wid = lax.axis_index("s") * _NC + lax.axis_index("c")
        base = wid * n_per_w
        pltpu.async_copy(idx_hbm.at[pl.ds(base, n_per_w)], idx_v, isem).wait()

        @pl.loop(0, n_chunks, step=_NBUF)
        def _(r):
            for j in range(_NBUF):
                c = r + j
                idx_slice = idx_v.at[pl.ds(c * _CHUNK, _CHUNK)]
                out_slice = out_hbm.at[pl.ds(base + c * _CHUNK, _CHUNK)]

                @pl.when(r > 0)
                def _():
                    pltpu.make_async_copy(bufs[j], out_slice, wsems[j]).wait()

                pltpu.make_async_copy(
                    table_hbm.at[idx_slice], bufs[j], gsems[j]
                ).start()

            for j in range(_NBUF):
                c = r + j
                idx_slice = idx_v.at[pl.ds(c * _CHUNK, _CHUNK)]
                out_slice = out_hbm.at[pl.ds(base + c * _CHUNK, _CHUNK)]
                pltpu.make_async_copy(
                    table_hbm.at[idx_slice], bufs[j], gsems[j]
                ).wait()
                pltpu.make_async_copy(bufs[j], out_slice, wsems[j]).start()

        for j in range(_NBUF):
            c = n_chunks - _NBUF + j
            out_slice = out_hbm.at[pl.ds(base + c * _CHUNK, _CHUNK)]
            pltpu.make_async_copy(bufs[j], out_slice, wsems[j]).wait()

    out = gather_kernel(table, indices)
    return out.reshape(x.shape[1], x.shape[0], embed_dim).transpose(1, 0, 2)


# speedup vs baseline: 1.0302x; 1.0302x over previous
import jax
import jax.numpy as jnp
from jax import lax
from jax.experimental import pallas as pl
from jax.experimental.pallas import tpu as pltpu
from jax.experimental.pallas import tpu_sc as plsc

_NC = 2
_NS = 16
_NW = _NC * _NS
_CHUNK = 64
_NBUF = 5


def kernel(x, table):
    num_indices = x.shape[0] * x.shape[1]
    embed_dim = table.shape[1]
    n_per_w = num_indices // _NW
    n_chunks = n_per_w // _CHUNK
    indices = x.T.reshape(num_indices)

    mesh = plsc.VectorSubcoreMesh(core_axis_name="c", subcore_axis_name="s")

    @jax.jit
    @pl.kernel(
        out_type=jax.ShapeDtypeStruct((num_indices, embed_dim), jnp.float32),
        mesh=mesh,
        scratch_types=(
            [pltpu.VMEM((n_per_w,), jnp.int32)]
            + [pltpu.VMEM((_CHUNK, embed_dim), jnp.float32) for _ in range(_NBUF)]
            + [pltpu.SemaphoreType.DMA for _ in range(2 * _NBUF + 1)]
        ),
    )
    def gather_kernel(table_hbm, idx_hbm, out_hbm, idx_v, *bufs_and_sems):
        bufs = bufs_and_sems[:_NBUF]
        gsems = bufs_and_sems[_NBUF:2 * _NBUF]
        wsems = bufs_and_sems[2 * _NBUF:3 * _NBUF]
        isem = bufs_and_sems[3 * _NBUF]

        wid = lax.axis_index("s") * _NC + lax.axis_index("c")
        base = wid * n_per_w
        pltpu.async_copy(idx_hbm.at[pl.ds(base, n_per_w)], idx_v, isem).wait()

        @pl.loop(0, n_chunks, step=_NBUF)
        def _(r):
            for j in range(_NBUF):
                c = r + j
                idx_slice = idx_v.at[pl.ds(c * _CHUNK, _CHUNK)]
                out_slice = out_hbm.at[pl.ds(base + c * _CHUNK, _CHUNK)]

                @pl.when(r > 0)
                def _():
                    pltpu.make_async_copy(bufs[j], out_slice, wsems[j]).wait()

                pltpu.make_async_copy(
                    table_hbm.at[idx_slice], bufs[j], gsems[j]
                ).start()

            for j in range(_NBUF):
                c = r + j
                idx_slice = idx_v.at[pl.ds(c * _CHUNK, _CHUNK)]
                out_slice = out_hbm.at[pl.ds(base + c * _CHUNK, _CHUNK)]
                pltpu.make_async_copy(
                    table_hbm.at[idx_slice], bufs[j], gsems[j]
                ).wait()
                pltpu.make_async_copy(bufs[j], out_slice, wsems[j]).start()

        for j in range(_NBUF):
            c = n_chunks - _NBUF + j
            out_slice = out_hbm.at[pl.ds(base + c * _CHUNK, _CHUNK)]
            pltpu.make_async_copy(bufs[j], out_slice, wsems[j]).wait()

    out = gather_kernel(table, indices)
    return out.reshape(x.shape[1], x.shape[0], embed_dim).transpose(1, 0, 2)


# speedup vs baseline: 1.0306x; 1.0004x over previous
import jax
import jax.numpy as jnp
from jax import lax
from jax.experimental import pallas as pl
from jax.experimental.pallas import tpu as pltpu
from jax.experimental.pallas import tpu_sc as plsc

_NC = 2
_NS = 16
_NW = _NC * _NS
_CHUNK = 32
_NBUF = 8


def kernel(x, table):
    num_indices = x.shape[0] * x.shape[1]
    embed_dim = table.shape[1]
    n_per_w = num_indices // _NW
    n_chunks = n_per_w // _CHUNK
    indices = x.T.reshape(num_indices)

    mesh = plsc.VectorSubcoreMesh(core_axis_name="c", subcore_axis_name="s")

    @jax.jit
    @pl.kernel(
        out_type=jax.ShapeDtypeStruct((num_indices, embed_dim), jnp.float32),
        mesh=mesh,
        scratch_types=(
            [pltpu.VMEM((n_per_w,), jnp.int32)]
            + [pltpu.VMEM((_CHUNK, embed_dim), jnp.float32) for _ in range(_NBUF)]
            + [pltpu.SemaphoreType.DMA for _ in range(2 * _NBUF + 1)]
        ),
    )
    def gather_kernel(table_hbm, idx_hbm, out_hbm, idx_v, *bufs_and_sems):
        bufs = bufs_and_sems[:_NBUF]
        gsems = bufs_and_sems[_NBUF:2 * _NBUF]
        wsems = bufs_and_sems[2 * _NBUF:3 * _NBUF]
        isem = bufs_and_sems[3 * _NBUF]

        wid = lax.axis_index("s") * _NC + lax.axis_index("c")
        base = wid * n_per_w
        pltpu.async_copy(idx_hbm.at[pl.ds(base, n_per_w)], idx_v, isem).wait()

        @pl.loop(0, n_chunks, step=_NBUF)
        def _(r):
            for j in range(_NBUF):
                c = r + j
                idx_slice = idx_v.at[pl.ds(c * _CHUNK, _CHUNK)]
                out_slice = out_hbm.at[pl.ds(base + c * _CHUNK, _CHUNK)]

                @pl.when(r > 0)
                def _():
                    pltpu.make_async_copy(bufs[j], out_slice, wsems[j]).wait()

                pltpu.make_async_copy(
                    table_hbm.at[idx_slice], bufs[j], gsems[j]
                ).start()

            for j in range(_NBUF):
                c = r + j
                idx_slice = idx_v.at[pl.ds(c * _CHUNK, _CHUNK)]
                out_slice = out_hbm.at[pl.ds(base + c * _CHUNK, _CHUNK)]
                pltpu.make_async_copy(
                    table_hbm.at[idx_slice], bufs[j], gsems[j]
                ).wait()
                pltpu.make_async_copy(bufs[j], out_slice, wsems[j]).start()

        for j in range(_NBUF):
            c = n_chunks - _NBUF + j
            out_slice = out_hbm.at[pl.ds(base + c * _CHUNK, _CHUNK)]
            pltpu.make_async_copy(bufs[j], out_slice, wsems[j]).wait()

    out = gather_kernel(table, indices)
    return out.reshape(x.shape[1], x.shape[0], embed_dim).transpose(1, 0, 2)


# speedup vs baseline: 1.0337x; 1.0030x over previous
import jax
import jax.numpy as jnp
from jax import lax
from jax.experimental import pallas as pl
from jax.experimental.pallas import tpu as pltpu
from jax.experimental.pallas import tpu_sc as plsc

_NC = 2
_NS = 16
_NW = _NC * _NS
_CHUNK = 32
_NBUF = 10


def kernel(x, table):
    num_indices = x.shape[0] * x.shape[1]
    embed_dim = table.shape[1]
    n_per_w = num_indices // _NW
    n_chunks = n_per_w // _CHUNK
    indices = x.T.reshape(num_indices)

    mesh = plsc.VectorSubcoreMesh(core_axis_name="c", subcore_axis_name="s")

    @jax.jit
    @pl.kernel(
        out_type=jax.ShapeDtypeStruct((num_indices, embed_dim), jnp.float32),
        mesh=mesh,
        scratch_types=(
            [pltpu.VMEM((n_per_w,), jnp.int32)]
            + [pltpu.VMEM((_CHUNK, embed_dim), jnp.float32) for _ in range(_NBUF)]
            + [pltpu.SemaphoreType.DMA for _ in range(2 * _NBUF + 1)]
        ),
    )
    def gather_kernel(table_hbm, idx_hbm, out_hbm, idx_v, *bufs_and_sems):
        bufs = bufs_and_sems[:_NBUF]
        gsems = bufs_and_sems[_NBUF:2 * _NBUF]
        wsems = bufs_and_sems[2 * _NBUF:3 * _NBUF]
        isem = bufs_and_sems[3 * _NBUF]

        wid = lax.axis_index("s") * _NC + lax.axis_index("c")
        base = wid * n_per_w
        pltpu.async_copy(idx_hbm.at[pl.ds(base, n_per_w)], idx_v, isem).wait()

        @pl.loop(0, n_chunks, step=_NBUF)
        def _(r):
            for j in range(_NBUF):
                c = r + j
                idx_slice = idx_v.at[pl.ds(c * _CHUNK, _CHUNK)]
                out_slice = out_hbm.at[pl.ds(base + c * _CHUNK, _CHUNK)]

                @pl.when(r > 0)
                def _():
                    pltpu.make_async_copy(bufs[j], out_slice, wsems[j]).wait()

                pltpu.make_async_copy(
                    table_hbm.at[idx_slice], bufs[j], gsems[j]
                ).start()

            for j in range(_NBUF):
                c = r + j
                idx_slice = idx_v.at[pl.ds(c * _CHUNK, _CHUNK)]
                out_slice = out_hbm.at[pl.ds(base + c * _CHUNK, _CHUNK)]
                pltpu.make_async_copy(
                    table_hbm.at[idx_slice], bufs[j], gsems[j]
                ).wait()
                pltpu.make_async_copy(bufs[j], out_slice, wsems[j]).start()

        for j in range(_NBUF):
            c = n_chunks - _NBUF + j
            out_slice = out_hbm.at[pl.ds(base + c * _CHUNK, _CHUNK)]
            pltpu.make_async_copy(bufs[j], out_slice, wsems[j]).wait()

    out = gather_kernel(table, indices)
    return out.reshape(x.shape[1], x.shape[0], embed_dim).transpose(1, 0, 2)
